# baseline (speedup 1.0000x reference)
import jax
import jax.numpy as jnp
from jax.experimental import pallas as pl

B = 1024
D = 64
K = 100000
BLK_K = 2048


def _logits_kernel(x_ref, p_ref, o_ref):
    o_ref[...] = jax.lax.dot_general(
        x_ref[...],
        p_ref[...],
        dimension_numbers=(((1,), (1,)), ((), ())),
        preferred_element_type=jnp.float32,
    )


def kernel(x, proto):
    return pl.pallas_call(
        _logits_kernel,
        grid=(pl.cdiv(K, BLK_K),),
        in_specs=[
            pl.BlockSpec((B, D), lambda k: (0, 0)),
            pl.BlockSpec((BLK_K, D), lambda k: (k, 0)),
        ],
        out_specs=pl.BlockSpec((B, BLK_K), lambda k: (0, k)),
        out_shape=jax.ShapeDtypeStruct((B, K), jnp.float32),
    )(x, proto)
